# Initial kernel scaffold; baseline (speedup 1.0000x reference)
import jax
import jax.numpy as jnp
from jax.experimental import pallas as pl


def kernel(features, labels, prototypes):
    raise NotImplementedError("write your pallas kernel here")


# speedup vs baseline: 48.0949x; 48.0949x over previous
import jax
import jax.numpy as jnp
from jax import lax
from jax.experimental import pallas as pl
from jax.experimental.pallas import tpu as pltpu

N_CLS = 1000
FEAT_DIM = 512
BATCH = 1024
PROTO_M = 0.95
TEMPERATURE = 0.1
BASE_TEMPERATURE = 0.1


def _body(lab_ref, feat_ref, proto_ref, out_ref, pscr):
    pscr[...] = proto_ref[...]

    def step(i, carry):
        l = lab_ref[i]
        row = pscr[pl.ds(l, 1), :]
        f = feat_ref[pl.ds(i, 1), :]
        p = row * PROTO_M + f * (1.0 - PROTO_M)
        n = jnp.sqrt(jnp.sum(p * p))
        p = p / jnp.maximum(n, 1e-12)
        pscr[pl.ds(l, 1), :] = p
        return carry

    lax.fori_loop(0, BATCH, step, 0, unroll=False)

    protos = pscr[...]
    logits = lax.dot_general(
        protos, protos, (((1,), (1,)), ((), ())),
        preferred_element_type=jnp.float32) * (1.0 / TEMPERATURE)
    e = jnp.exp(logits)
    ri = lax.broadcasted_iota(jnp.int32, (N_CLS, N_CLS), 0)
    ci = lax.broadcasted_iota(jnp.int32, (N_CLS, N_CLS), 1)
    e = jnp.where(ri == ci, 0.0, e)
    s = jnp.sum(e, axis=1)
    mpn = jnp.log(s * (1.0 / (N_CLS - 1)))
    nan = jnp.isnan(mpn)
    total = jnp.sum(jnp.where(nan, 0.0, mpn))
    cnt = jnp.sum(jnp.where(nan, 0.0, 1.0))
    out_ref[0, 0] = (TEMPERATURE / BASE_TEMPERATURE) * total / cnt


def kernel(features, labels, prototypes):
    labels = labels.astype(jnp.int32)
    out = pl.pallas_call(
        _body,
        out_shape=jax.ShapeDtypeStruct((1, 1), jnp.float32),
        in_specs=[
            pl.BlockSpec(memory_space=pltpu.SMEM),
            pl.BlockSpec(memory_space=pltpu.VMEM),
            pl.BlockSpec(memory_space=pltpu.VMEM),
        ],
        out_specs=pl.BlockSpec(memory_space=pltpu.SMEM),
        scratch_shapes=[pltpu.VMEM((N_CLS, FEAT_DIM), jnp.float32)],
    )(labels, features, prototypes)
    return out[0, 0]


# speedup vs baseline: 51.1658x; 1.0639x over previous
import jax
import jax.numpy as jnp
from jax import lax
from jax.experimental import pallas as pl
from jax.experimental.pallas import tpu as pltpu
from jax.experimental.pallas import tpu_sc as plsc

N_CLS = 1000
FEAT_DIM = 512
BATCH = 1024
PROTO_M = 0.95
TEMPERATURE = 0.1
BASE_TEMPERATURE = 0.1

_PAD_CLS = 1024
_RPW = 32
_L = 16
_NCHUNK = BATCH // _L
_FCH = FEAT_DIM // _L


def _sc_body(lab_hbm, feat_hbm, proto_hbm, out_hbm, lab_v, blk_v, frow_v, tmp_v):
    wid = lax.axis_index("c") * 16 + lax.axis_index("s")
    lo = wid * _RPW
    base = wid * (_RPW * FEAT_DIM)

    pltpu.sync_copy(lab_hbm, lab_v)
    pltpu.sync_copy(proto_hbm.at[pl.ds(base, _RPW * FEAT_DIM)], blk_v)

    def do_update(l_scalar, i_scalar):
        rb = (l_scalar - lo) * FEAT_DIM
        pltpu.sync_copy(feat_hbm.at[pl.ds(i_scalar * FEAT_DIM, FEAT_DIM)],
                        frow_v)

        def ema(c, acc):
            pr = blk_v[pl.ds(rb + c * _L, _L)]
            f = frow_v[pl.ds(c * _L, _L)]
            p = pr * PROTO_M + f * (1.0 - PROTO_M)
            tmp_v[pl.ds(c * _L, _L)] = p
            return acc + p * p

        acc = lax.fori_loop(0, _FCH, ema, jnp.zeros((_L,), jnp.float32),
                            unroll=False)
        s = acc[0]
        for t in range(1, _L):
            s = s + acc[t]
        si = lax.bitcast_convert_type(s, jnp.int32)
        yi = jnp.int32(0x5F3759DF) - (si >> 1)
        y = lax.bitcast_convert_type(yi, jnp.float32)
        for _ in range(3):
            y = y * (1.5 - 0.5 * s * y * y)
        y = jnp.minimum(y, jnp.float32(1e12))
        inv = jnp.full((_L,), 1.0, jnp.float32) * y

        def scale(c, c2):
            blk_v[pl.ds(rb + c * _L, _L)] = tmp_v[pl.ds(c * _L, _L)] * inv
            return c2

        lax.fori_loop(0, _FCH, scale, 0, unroll=False)

    def chunk_step(k, carry):
        v = lab_v[pl.ds(k * _L, _L)]
        for t in range(_L):
            l = v[t]
            hit = (l >= lo) & (l < lo + _RPW)

            @pl.when(hit)
            def _u(l=l, t=t, k=k):
                do_update(l, k * _L + t)
        return carry

    lax.fori_loop(0, _NCHUNK, chunk_step, 0, unroll=False)
    pltpu.sync_copy(blk_v, out_hbm.at[pl.ds(base, _RPW * FEAT_DIM)])


def _sc_update(labels, feat_flat, proto_flat):
    mesh = plsc.VectorSubcoreMesh(core_axis_name="c", subcore_axis_name="s")
    return pl.kernel(
        _sc_body,
        out_type=jax.ShapeDtypeStruct((_PAD_CLS * FEAT_DIM,), jnp.float32),
        mesh=mesh,
        scratch_types=[
            pltpu.VMEM((BATCH,), jnp.int32),
            pltpu.VMEM((_RPW * FEAT_DIM,), jnp.float32),
            pltpu.VMEM((FEAT_DIM,), jnp.float32),
            pltpu.VMEM((FEAT_DIM,), jnp.float32),
        ],
    )(labels, feat_flat, proto_flat)


def _loss_body(proto_ref, out_ref):
    protos = proto_ref[0:N_CLS, :]
    logits = lax.dot_general(
        protos, protos, (((1,), (1,)), ((), ())),
        preferred_element_type=jnp.float32) * (1.0 / TEMPERATURE)
    e = jnp.exp(logits)
    ri = lax.broadcasted_iota(jnp.int32, (N_CLS, N_CLS), 0)
    ci = lax.broadcasted_iota(jnp.int32, (N_CLS, N_CLS), 1)
    e = jnp.where(ri == ci, 0.0, e)
    s = jnp.sum(e, axis=1)
    mpn = jnp.log(s * (1.0 / (N_CLS - 1)))
    nan = jnp.isnan(mpn)
    total = jnp.sum(jnp.where(nan, 0.0, mpn))
    cnt = jnp.sum(jnp.where(nan, 0.0, 1.0))
    out_ref[0, 0] = (TEMPERATURE / BASE_TEMPERATURE) * total / cnt


def kernel(features, labels, prototypes):
    labels = labels.astype(jnp.int32)
    proto_flat = jnp.zeros((_PAD_CLS, FEAT_DIM), jnp.float32).at[:N_CLS].set(
        prototypes).reshape(-1)
    upd = _sc_update(labels, features.reshape(-1), proto_flat)
    upd = upd.reshape(_PAD_CLS, FEAT_DIM)
    out = pl.pallas_call(
        _loss_body,
        out_shape=jax.ShapeDtypeStruct((1, 1), jnp.float32),
        in_specs=[pl.BlockSpec(memory_space=pltpu.VMEM)],
        out_specs=pl.BlockSpec(memory_space=pltpu.SMEM),
    )(upd)
    return out[0, 0]


# speedup vs baseline: 124.4924x; 2.4331x over previous
import jax
import jax.numpy as jnp
from jax import lax
from jax.experimental import pallas as pl
from jax.experimental.pallas import tpu as pltpu
from jax.experimental.pallas import tpu_sc as plsc

N_CLS = 1000
FEAT_DIM = 512
BATCH = 1024
PROTO_M = 0.95
TEMPERATURE = 0.1
BASE_TEMPERATURE = 0.1

_PAD_CLS = 1024
_RPW = 32
_NW = 32
_L = 16
_FCH = FEAT_DIM // _L
_BP = BATCH + _NW * (_L - 1) + _L


def _prep_body(lab_row_ref, lab_col_ref, feat_ref, fperm_ref, lperm_ref,
               off_ref, cnt_ref):
    lr = lab_row_ref[...].astype(jnp.float32)
    lc = lab_col_ref[...].astype(jnp.float32)
    wr = jnp.floor(lr * (1.0 / _RPW))
    wc = jnp.floor(lc * (1.0 / _RPW))

    ii = lax.broadcasted_iota(jnp.int32, (BATCH, BATCH), 0)
    jj = lax.broadcasted_iota(jnp.int32, (BATCH, BATCH), 1)
    same_w = (wc == wr).astype(jnp.float32)
    before = (jj < ii).astype(jnp.float32)
    slot = jnp.sum(same_w * before, axis=1, keepdims=True)

    wi = lax.broadcasted_iota(jnp.int32, (_NW, BATCH), 0).astype(jnp.float32)
    wonehot = (wi == wr).astype(jnp.float32)
    cnt = jnp.sum(wonehot, axis=1, keepdims=True)
    cnt_al = jnp.ceil(cnt * (1.0 / _L)) * _L
    tri = (lax.broadcasted_iota(jnp.int32, (_NW, _NW), 1)
           < lax.broadcasted_iota(jnp.int32, (_NW, _NW), 0)).astype(
               jnp.float32)
    off = lax.dot_general(tri, cnt_al, (((1,), (0,)), ((), ())),
                          preferred_element_type=jnp.float32)

    wrow = lax.broadcasted_iota(jnp.int32, (1, _NW), 1).astype(jnp.float32)
    myoff = lax.dot_general(
        (wc == wrow).astype(jnp.float32), off,
        (((1,), (0,)), ((), ())), preferred_element_type=jnp.float32)
    pos = myoff + slot
    sel = (lax.broadcasted_iota(jnp.int32, (BATCH, _BP), 1).astype(
        jnp.float32) == pos).astype(jnp.float32)

    fperm_ref[...] = lax.dot_general(
        sel, feat_ref[...], (((0,), (0,)), ((), ())),
        preferred_element_type=jnp.float32)
    lperm_ref[...] = lax.dot_general(
        lr, sel, (((1,), (0,)), ((), ())),
        preferred_element_type=jnp.float32).astype(jnp.int32)
    off_ref[...] = jnp.broadcast_to(off, (_NW, _L)).astype(jnp.int32)
    cnt_ref[...] = jnp.broadcast_to(cnt, (_NW, _L)).astype(jnp.int32)


def _prep(labels, features):
    return pl.pallas_call(
        _prep_body,
        out_shape=[
            jax.ShapeDtypeStruct((_BP, FEAT_DIM), jnp.float32),
            jax.ShapeDtypeStruct((1, _BP), jnp.int32),
            jax.ShapeDtypeStruct((_NW, _L), jnp.int32),
            jax.ShapeDtypeStruct((_NW, _L), jnp.int32),
        ],
        in_specs=[
            pl.BlockSpec(memory_space=pltpu.VMEM),
            pl.BlockSpec(memory_space=pltpu.VMEM),
            pl.BlockSpec(memory_space=pltpu.VMEM),
        ],
        out_specs=[
            pl.BlockSpec(memory_space=pltpu.VMEM),
            pl.BlockSpec(memory_space=pltpu.VMEM),
            pl.BlockSpec(memory_space=pltpu.VMEM),
            pl.BlockSpec(memory_space=pltpu.VMEM),
        ],
    )(labels.reshape(1, BATCH), labels.reshape(BATCH, 1), features)


def _sc_body(lab_hbm, ofs_hbm, cnt_hbm, feat_hbm, proto_hbm, out_hbm,
             lab_v, ofs_v, cnt_v, blk_v, fbuf_v, tmp_v):
    wid = lax.axis_index("c") * 16 + lax.axis_index("s")
    lo = wid * _RPW
    base = wid * (_RPW * FEAT_DIM)

    pltpu.sync_copy(lab_hbm, lab_v)
    pltpu.sync_copy(ofs_hbm.at[pl.ds(wid * _L, _L)], ofs_v)
    pltpu.sync_copy(cnt_hbm.at[pl.ds(wid * _L, _L)], cnt_v)
    pltpu.sync_copy(proto_hbm.at[pl.ds(base, _RPW * FEAT_DIM)], blk_v)

    o = ofs_v[pl.ds(0, _L)][0]
    n = cnt_v[pl.ds(0, _L)][0]
    nch = (n + (_L - 1)) // _L

    def do_update(l_scalar, frow_base):
        rb = (l_scalar - lo) * FEAT_DIM

        def ema(c, acc):
            pr = blk_v[pl.ds(rb + c * _L, _L)]
            f = fbuf_v[pl.ds(frow_base + c * _L, _L)]
            p = pr * PROTO_M + f * (1.0 - PROTO_M)
            tmp_v[pl.ds(c * _L, _L)] = p
            return acc + p * p

        acc = lax.fori_loop(0, _FCH, ema, jnp.zeros((_L,), jnp.float32),
                            unroll=8)
        s = acc[0]
        for t in range(1, _L):
            s = s + acc[t]
        si = lax.bitcast_convert_type(s, jnp.int32)
        yi = jnp.int32(0x5F3759DF) - (si >> 1)
        y = lax.bitcast_convert_type(yi, jnp.float32)
        for _ in range(3):
            y = y * (1.5 - 0.5 * s * y * y)
        y = jnp.minimum(y, jnp.float32(1e12))
        inv = jnp.full((_L,), 1.0, jnp.float32) * y

        def scale(c, c2):
            blk_v[pl.ds(rb + c * _L, _L)] = tmp_v[pl.ds(c * _L, _L)] * inv
            return c2

        lax.fori_loop(0, _FCH, scale, 0, unroll=8)

    def chunk_step(k, carry):
        pltpu.sync_copy(
            feat_hbm.at[pl.ds((o + k * _L) * FEAT_DIM, _L * FEAT_DIM)],
            fbuf_v)
        v = lab_v[pl.ds(o + k * _L, _L)]
        for t in range(_L):
            @pl.when(k * _L + t < n)
            def _u(t=t):
                do_update(v[t], t * FEAT_DIM)
        return carry

    lax.fori_loop(0, nch, chunk_step, 0, unroll=False)
    pltpu.sync_copy(blk_v, out_hbm.at[pl.ds(base, _RPW * FEAT_DIM)])


def _sc_update(lab_perm, offs, cnts, feat_perm_flat, proto_flat):
    mesh = plsc.VectorSubcoreMesh(core_axis_name="c", subcore_axis_name="s")
    return pl.kernel(
        _sc_body,
        out_type=jax.ShapeDtypeStruct((_PAD_CLS * FEAT_DIM,), jnp.float32),
        mesh=mesh,
        scratch_types=[
            pltpu.VMEM((_BP,), jnp.int32),
            pltpu.VMEM((_L,), jnp.int32),
            pltpu.VMEM((_L,), jnp.int32),
            pltpu.VMEM((_RPW * FEAT_DIM,), jnp.float32),
            pltpu.VMEM((_L * FEAT_DIM,), jnp.float32),
            pltpu.VMEM((FEAT_DIM,), jnp.float32),
        ],
    )(lab_perm, offs, cnts, feat_perm_flat, proto_flat)


def _loss_body(proto_ref, out_ref):
    protos = proto_ref[0:N_CLS, :]
    logits = lax.dot_general(
        protos, protos, (((1,), (1,)), ((), ())),
        preferred_element_type=jnp.float32) * (1.0 / TEMPERATURE)
    e = jnp.exp(logits)
    ri = lax.broadcasted_iota(jnp.int32, (N_CLS, N_CLS), 0)
    ci = lax.broadcasted_iota(jnp.int32, (N_CLS, N_CLS), 1)
    e = jnp.where(ri == ci, 0.0, e)
    s = jnp.sum(e, axis=1)
    mpn = jnp.log(s * (1.0 / (N_CLS - 1)))
    nan = jnp.isnan(mpn)
    total = jnp.sum(jnp.where(nan, 0.0, mpn))
    cnt = jnp.sum(jnp.where(nan, 0.0, 1.0))
    out_ref[0, 0] = (TEMPERATURE / BASE_TEMPERATURE) * total / cnt


def kernel(features, labels, prototypes):
    labels = labels.astype(jnp.int32)
    fperm, lperm, offs, cnts = _prep(labels, features)
    proto_flat = jnp.zeros((_PAD_CLS, FEAT_DIM), jnp.float32).at[:N_CLS].set(
        prototypes).reshape(-1)
    upd = _sc_update(lperm.reshape(-1), offs.reshape(-1), cnts.reshape(-1),
                     fperm.reshape(-1), proto_flat)
    upd = upd.reshape(_PAD_CLS, FEAT_DIM)
    out = pl.pallas_call(
        _loss_body,
        out_shape=jax.ShapeDtypeStruct((1, 1), jnp.float32),
        in_specs=[pl.BlockSpec(memory_space=pltpu.VMEM)],
        out_specs=pl.BlockSpec(memory_space=pltpu.SMEM),
    )(upd)
    return out[0, 0]


# speedup vs baseline: 185.8877x; 1.4932x over previous
import jax
import jax.numpy as jnp
from jax import lax
from jax.experimental import pallas as pl
from jax.experimental.pallas import tpu as pltpu
from jax.experimental.pallas import tpu_sc as plsc

N_CLS = 1000
FEAT_DIM = 512
BATCH = 1024
PROTO_M = 0.95
TEMPERATURE = 0.1
BASE_TEMPERATURE = 0.1

_RPW = 32
_NW = 32
_TAIL = N_CLS - (_NW - 1) * _RPW
_L = 16
_FCH = FEAT_DIM // _L
_BP = BATCH + _NW * (_L - 1) + _L


def _prep_body(lab_row_ref, lab_col_ref, feat_ref, fperm_ref, labbc_ref,
               ocnt_ref):
    lr = lab_row_ref[...].astype(jnp.float32)
    lc = lab_col_ref[...].astype(jnp.float32)
    wr = jnp.floor(lr * (1.0 / _RPW))
    wc = jnp.floor(lc * (1.0 / _RPW))

    ii = lax.broadcasted_iota(jnp.int32, (BATCH, BATCH), 0)
    jj = lax.broadcasted_iota(jnp.int32, (BATCH, BATCH), 1)
    same_w = (wc == wr).astype(jnp.float32)
    before = (jj < ii).astype(jnp.float32)
    slot = jnp.sum(same_w * before, axis=1, keepdims=True)

    wi = lax.broadcasted_iota(jnp.int32, (_NW, BATCH), 0).astype(jnp.float32)
    wonehot = (wi == wr).astype(jnp.float32)
    cnt = jnp.sum(wonehot, axis=1, keepdims=True)
    cnt_al = jnp.ceil(cnt * (1.0 / _L)) * _L
    tri = (lax.broadcasted_iota(jnp.int32, (_NW, _NW), 1)
           < lax.broadcasted_iota(jnp.int32, (_NW, _NW), 0)).astype(
               jnp.float32)
    off = lax.dot_general(tri, cnt_al, (((1,), (0,)), ((), ())),
                          preferred_element_type=jnp.float32)

    wrow = lax.broadcasted_iota(jnp.int32, (1, _NW), 1).astype(jnp.float32)
    myoff = lax.dot_general(
        (wc == wrow).astype(jnp.float32), off,
        (((1,), (0,)), ((), ())), preferred_element_type=jnp.float32)
    pos = myoff + slot
    sel = (lax.broadcasted_iota(jnp.int32, (BATCH, _BP), 1).astype(
        jnp.float32) == pos).astype(jnp.float32)

    fperm_ref[...] = lax.dot_general(
        sel, feat_ref[...], (((0,), (0,)), ((), ())),
        preferred_element_type=jnp.float32)
    lab_bc = jnp.broadcast_to(lc, (BATCH, _L))
    labbc_ref[...] = lax.dot_general(
        sel, lab_bc, (((0,), (0,)), ((), ())),
        preferred_element_type=jnp.float32).astype(jnp.int32)
    li = lax.broadcasted_iota(jnp.int32, (_NW, _L), 1)
    ocnt = (jnp.broadcast_to(off, (_NW, _L)) * (li == 0)
            + jnp.broadcast_to(cnt, (_NW, _L)) * (li == 1))
    ocnt_ref[...] = ocnt.astype(jnp.int32)


def _prep(labels, features):
    return pl.pallas_call(
        _prep_body,
        out_shape=[
            jax.ShapeDtypeStruct((_BP, FEAT_DIM), jnp.float32),
            jax.ShapeDtypeStruct((_BP, _L), jnp.int32),
            jax.ShapeDtypeStruct((_NW, _L), jnp.int32),
        ],
        in_specs=[
            pl.BlockSpec(memory_space=pltpu.VMEM),
            pl.BlockSpec(memory_space=pltpu.VMEM),
            pl.BlockSpec(memory_space=pltpu.VMEM),
        ],
        out_specs=[
            pl.BlockSpec(memory_space=pltpu.VMEM),
            pl.BlockSpec(memory_space=pltpu.VMEM),
            pl.BlockSpec(memory_space=pltpu.VMEM),
        ],
    )(labels.reshape(1, BATCH), labels.reshape(BATCH, 1), features)


def _sc_body(labbc_hbm, ocnt_hbm, feat_hbm, proto_hbm, out_hbm,
             ocnt_v, blk_v, fbuf_v, labseg_v, scl_v, sem1, sem2):
    wid = lax.axis_index("c") * 16 + lax.axis_index("s")
    lo = pl.multiple_of(wid * _RPW, _RPW)
    rpw = jnp.minimum(jnp.int32(_RPW), jnp.int32(N_CLS) - wid * _RPW)

    cdesc = pltpu.async_copy(ocnt_hbm.at[pl.ds(wid * _L, _L)], ocnt_v, sem1)

    @pl.when(wid < _NW - 1)
    def _full_blk():
        pltpu.sync_copy(proto_hbm.at[pl.ds(lo, _RPW)], blk_v)

    @pl.when(wid == _NW - 1)
    def _tail_blk():
        pltpu.sync_copy(proto_hbm.at[pl.ds(lo, _TAIL)],
                        blk_v.at[pl.ds(0, _TAIL)])

    cdesc.wait()
    v = ocnt_v[pl.ds(0, _L)]
    o = v[0]
    n = v[1]
    nch = (n + (_L - 1)) >> 4

    ones = jnp.full((_L,), 1.0, jnp.float32)

    def init_scl(r, carry):
        scl_v[r, pl.ds(0, _L)] = ones
        return carry

    lax.fori_loop(0, _RPW, init_scl, 0, unroll=4)

    lane = lax.iota(jnp.int32, _L)

    def do_update(l_scalar, j):
        r = l_scalar - lo
        g = scl_v[r, pl.ds(0, _L)]
        mg = g * PROTO_M

        def ema(ci, accs):
            a0, a1, a2, a3 = accs
            out = []
            for u, a in ((0, a0), (1, a1), (2, a2), (3, a3)):
                cc = ci * 4 + u
                p = (blk_v[r, pl.ds(cc * _L, _L)] * mg
                     + fbuf_v[j, pl.ds(cc * _L, _L)] * (1.0 - PROTO_M))
                blk_v[r, pl.ds(cc * _L, _L)] = p
                out.append(a + p * p)
            return tuple(out)

        z = jnp.zeros((_L,), jnp.float32)
        a0, a1, a2, a3 = lax.fori_loop(0, _FCH // 4, ema, (z, z, z, z),
                                       unroll=2)
        s = (a0 + a1) + (a2 + a3)
        for sh in (8, 4, 2, 1):
            s = s + s[lane ^ sh]
        si = lax.bitcast_convert_type(s, jnp.int32)
        yi = jnp.int32(0x5F3759DF) - (si >> 1)
        y = lax.bitcast_convert_type(yi, jnp.float32)
        for _ in range(3):
            y = y * (1.5 - 0.5 * s * y * y)
        scl_v[r, pl.ds(0, _L)] = jnp.minimum(y, jnp.float32(1e12))

    def chunk_step(k, carry):
        row = pl.multiple_of(o + k * _L, _L)
        c1 = pltpu.async_copy(feat_hbm.at[pl.ds(row, _L)], fbuf_v, sem1)
        c2 = pltpu.async_copy(labbc_hbm.at[pl.ds(row, _L)], labseg_v, sem2)
        c1.wait()
        c2.wait()
        m = jnp.minimum(n - k * _L, _L)

        def samp(j, c2_):
            l = labseg_v[j, pl.ds(0, _L)][0]
            do_update(l, j)
            return c2_

        lax.fori_loop(0, m, samp, 0)
        return carry

    lax.fori_loop(0, nch, chunk_step, 0)

    def wb_row(r, carry):
        g = scl_v[r, pl.ds(0, _L)]

        @pl.when(g[0] != 1.0)
        def _scale():
            def sc_chunk(ci, c2):
                for u in range(4):
                    cc = ci * 4 + u
                    blk_v[r, pl.ds(cc * _L, _L)] = (
                        blk_v[r, pl.ds(cc * _L, _L)] * g)
                return c2

            lax.fori_loop(0, _FCH // 4, sc_chunk, 0, unroll=2)
        return carry

    lax.fori_loop(0, rpw, wb_row, 0)

    @pl.when(wid < _NW - 1)
    def _full_out():
        pltpu.sync_copy(blk_v, out_hbm.at[pl.ds(lo, _RPW)])

    @pl.when(wid == _NW - 1)
    def _tail_out():
        pltpu.sync_copy(blk_v.at[pl.ds(0, _TAIL)],
                        out_hbm.at[pl.ds(lo, _TAIL)])


def _sc_update(labbc, ocnt, fperm, prototypes):
    mesh = plsc.VectorSubcoreMesh(core_axis_name="c", subcore_axis_name="s")
    return pl.kernel(
        _sc_body,
        out_type=jax.ShapeDtypeStruct((N_CLS, FEAT_DIM), jnp.float32),
        mesh=mesh,
        scratch_types=[
            pltpu.VMEM((_L,), jnp.int32),
            pltpu.VMEM((_RPW, FEAT_DIM), jnp.float32),
            pltpu.VMEM((_L, FEAT_DIM), jnp.float32),
            pltpu.VMEM((_L, _L), jnp.int32),
            pltpu.VMEM((_RPW, _L), jnp.float32),
            pltpu.SemaphoreType.DMA,
            pltpu.SemaphoreType.DMA,
        ],
    )(labbc, ocnt.reshape(-1), fperm, prototypes)


def _loss_body(proto_ref, out_ref):
    protos = proto_ref[...]
    logits = lax.dot_general(
        protos, protos, (((1,), (1,)), ((), ())),
        preferred_element_type=jnp.float32) * (1.0 / TEMPERATURE)
    e = jnp.exp(logits)
    ri = lax.broadcasted_iota(jnp.int32, (N_CLS, N_CLS), 0)
    ci = lax.broadcasted_iota(jnp.int32, (N_CLS, N_CLS), 1)
    e = jnp.where(ri == ci, 0.0, e)
    s = jnp.sum(e, axis=1)
    mpn = jnp.log(s * (1.0 / (N_CLS - 1)))
    nan = jnp.isnan(mpn)
    total = jnp.sum(jnp.where(nan, 0.0, mpn))
    cnt = jnp.sum(jnp.where(nan, 0.0, 1.0))
    out_ref[0, 0] = (TEMPERATURE / BASE_TEMPERATURE) * total / cnt


def kernel(features, labels, prototypes):
    labels = labels.astype(jnp.int32)
    fperm, labbc, ocnt = _prep(labels, features)
    upd = _sc_update(labbc, ocnt, fperm, prototypes)
    out = pl.pallas_call(
        _loss_body,
        out_shape=jax.ShapeDtypeStruct((1, 1), jnp.float32),
        in_specs=[pl.BlockSpec(memory_space=pltpu.VMEM)],
        out_specs=pl.BlockSpec(memory_space=pltpu.SMEM),
    )(upd)
    return out[0, 0]


# speedup vs baseline: 201.3033x; 1.0829x over previous
import jax
import jax.numpy as jnp
from jax import lax
from jax.experimental import pallas as pl
from jax.experimental.pallas import tpu as pltpu
from jax.experimental.pallas import tpu_sc as plsc

N_CLS = 1000
FEAT_DIM = 512
BATCH = 1024
PROTO_M = 0.95
TEMPERATURE = 0.1
BASE_TEMPERATURE = 0.1

_RPW = 32
_NW = 32
_TAIL = N_CLS - (_NW - 1) * _RPW
_L = 16
_FCH = FEAT_DIM // _L
_BP = BATCH + _NW * (_L - 1) + _L


def _prep_body(lab_row_ref, lab_col_ref, feat_ref, fperm_ref, labbc_ref,
               ocnt_ref):
    lr = lab_row_ref[...].astype(jnp.float32)
    lc = lab_col_ref[...].astype(jnp.float32)
    wr = jnp.floor(lr * (1.0 / _RPW))
    wc = jnp.floor(lc * (1.0 / _RPW))

    ii = lax.broadcasted_iota(jnp.int32, (BATCH, BATCH), 0)
    jj = lax.broadcasted_iota(jnp.int32, (BATCH, BATCH), 1)
    same_w = (wc == wr).astype(jnp.float32)
    before = (jj < ii).astype(jnp.float32)
    slot = jnp.sum(same_w * before, axis=1, keepdims=True)

    wi = lax.broadcasted_iota(jnp.int32, (_NW, BATCH), 0).astype(jnp.float32)
    wonehot = (wi == wr).astype(jnp.float32)
    cnt = jnp.sum(wonehot, axis=1, keepdims=True)
    cnt_al = jnp.ceil(cnt * (1.0 / _L)) * _L
    tri = (lax.broadcasted_iota(jnp.int32, (_NW, _NW), 1)
           < lax.broadcasted_iota(jnp.int32, (_NW, _NW), 0)).astype(
               jnp.float32)
    off = lax.dot_general(tri, cnt_al, (((1,), (0,)), ((), ())),
                          preferred_element_type=jnp.float32)

    wrow = lax.broadcasted_iota(jnp.int32, (1, _NW), 1).astype(jnp.float32)
    myoff = lax.dot_general(
        (wc == wrow).astype(jnp.float32), off,
        (((1,), (0,)), ((), ())), preferred_element_type=jnp.float32)
    pos = myoff + slot
    sel = (lax.broadcasted_iota(jnp.int32, (BATCH, _BP), 1).astype(
        jnp.float32) == pos).astype(jnp.float32)

    fperm_ref[...] = lax.dot_general(
        sel, feat_ref[...] * jnp.float32(1.0 - PROTO_M),
        (((0,), (0,)), ((), ())),
        preferred_element_type=jnp.float32)
    lab_bc = jnp.broadcast_to(lc, (BATCH, _L))
    labbc_ref[...] = lax.dot_general(
        sel, lab_bc, (((0,), (0,)), ((), ())),
        preferred_element_type=jnp.float32).astype(jnp.int32)
    li = lax.broadcasted_iota(jnp.int32, (_NW, _L), 1)
    ocnt = (jnp.broadcast_to(off, (_NW, _L)) * (li == 0)
            + jnp.broadcast_to(cnt, (_NW, _L)) * (li == 1))
    ocnt_ref[...] = ocnt.astype(jnp.int32)


def _prep(labels, features):
    return pl.pallas_call(
        _prep_body,
        out_shape=[
            jax.ShapeDtypeStruct((_BP, FEAT_DIM), jnp.float32),
            jax.ShapeDtypeStruct((_BP, _L), jnp.int32),
            jax.ShapeDtypeStruct((_NW, _L), jnp.int32),
        ],
        in_specs=[
            pl.BlockSpec(memory_space=pltpu.VMEM),
            pl.BlockSpec(memory_space=pltpu.VMEM),
            pl.BlockSpec(memory_space=pltpu.VMEM),
        ],
        out_specs=[
            pl.BlockSpec(memory_space=pltpu.VMEM),
            pl.BlockSpec(memory_space=pltpu.VMEM),
            pl.BlockSpec(memory_space=pltpu.VMEM),
        ],
    )(labels.reshape(1, BATCH), labels.reshape(BATCH, 1), features)


def _sc_body(labbc_hbm, ocnt_hbm, feat_hbm, proto_hbm, out_hbm, scl_hbm,
             ocnt_v, blk_v, fbuf_v, labseg_v, scl_v,
             sem1, semf0, semf1):
    wid = lax.axis_index("c") * 16 + lax.axis_index("s")
    lo = pl.multiple_of(wid * _RPW, _RPW)
    rpw = jnp.minimum(jnp.int32(_RPW), jnp.int32(N_CLS) - wid * _RPW)

    cdesc = pltpu.async_copy(ocnt_hbm.at[pl.ds(wid * _L, _L)], ocnt_v, sem1)

    @pl.when(wid < _NW - 1)
    def _full_blk():
        pltpu.sync_copy(proto_hbm.at[pl.ds(lo, _RPW)], blk_v)

    @pl.when(wid == _NW - 1)
    def _tail_blk():
        pltpu.sync_copy(proto_hbm.at[pl.ds(lo, _TAIL)],
                        blk_v.at[pl.ds(0, _TAIL)])

    cdesc.wait()
    v = ocnt_v[pl.ds(0, _L)]
    o = v[0]
    n = v[1]
    nch = (n + (_L - 1)) >> 4

    row0 = pl.multiple_of(o, _L)
    pltpu.async_copy(feat_hbm.at[pl.ds(row0, _L)],
                     fbuf_v.at[pl.ds(0, _L)], semf0)
    pltpu.async_copy(labbc_hbm.at[pl.ds(row0, _L)],
                     labseg_v.at[pl.ds(0, _L)], semf0)

    ones = jnp.full((_L,), 1.0, jnp.float32)

    def init_scl(r, carry):
        scl_v[r, pl.ds(0, _L)] = ones
        return carry

    lax.fori_loop(0, _RPW, init_scl, 0, unroll=4)

    lane = lax.iota(jnp.int32, _L)

    def do_update(l_scalar, j):
        r = l_scalar - lo
        g = scl_v[r, pl.ds(0, _L)]
        mg = g * PROTO_M

        def ema(ci, accs):
            a0, a1, a2, a3 = accs
            out = []
            for u, a in ((0, a0), (1, a1), (2, a2), (3, a3)):
                cc = ci * 4 + u
                p = (blk_v[r, pl.ds(cc * _L, _L)] * mg
                     + fbuf_v[j, pl.ds(cc * _L, _L)])
                blk_v[r, pl.ds(cc * _L, _L)] = p
                out.append(a + p * p)
            return tuple(out)

        z = jnp.zeros((_L,), jnp.float32)
        a0, a1, a2, a3 = lax.fori_loop(0, _FCH // 4, ema, (z, z, z, z),
                                       unroll=2)
        s = (a0 + a1) + (a2 + a3)
        for sh in (8, 4, 2, 1):
            s = s + s[lane ^ sh]
        si = lax.bitcast_convert_type(s, jnp.int32)
        yi = jnp.int32(0x5F3759DF) - (si >> 1)
        y = lax.bitcast_convert_type(yi, jnp.float32)
        for _ in range(3):
            y = y * (1.5 - 0.5 * s * y * y)
        scl_v[r, pl.ds(0, _L)] = jnp.minimum(y, jnp.float32(1e12))

    def chunk_step(k, carry):
        even = (k & 1) == 0
        nrow = pl.multiple_of(o + (k + 1) * _L, _L)
        crow = pl.multiple_of(o + k * _L, _L)

        @pl.when(even)
        def _issue_odd():
            pltpu.async_copy(feat_hbm.at[pl.ds(nrow, _L)],
                             fbuf_v.at[pl.ds(_L, _L)], semf1)
            pltpu.async_copy(labbc_hbm.at[pl.ds(nrow, _L)],
                             labseg_v.at[pl.ds(_L, _L)], semf1)

        @pl.when(jnp.logical_not(even))
        def _issue_even():
            pltpu.async_copy(feat_hbm.at[pl.ds(nrow, _L)],
                             fbuf_v.at[pl.ds(0, _L)], semf0)
            pltpu.async_copy(labbc_hbm.at[pl.ds(nrow, _L)],
                             labseg_v.at[pl.ds(0, _L)], semf0)

        @pl.when(even)
        def _wait_even():
            pltpu.make_async_copy(feat_hbm.at[pl.ds(crow, _L)],
                                  fbuf_v.at[pl.ds(0, _L)], semf0).wait()
            pltpu.make_async_copy(labbc_hbm.at[pl.ds(crow, _L)],
                                  labseg_v.at[pl.ds(0, _L)], semf0).wait()

        @pl.when(jnp.logical_not(even))
        def _wait_odd():
            pltpu.make_async_copy(feat_hbm.at[pl.ds(crow, _L)],
                                  fbuf_v.at[pl.ds(_L, _L)], semf1).wait()
            pltpu.make_async_copy(labbc_hbm.at[pl.ds(crow, _L)],
                                  labseg_v.at[pl.ds(_L, _L)], semf1).wait()

        base = (k & 1) * _L
        m = jnp.minimum(n - k * _L, _L)

        def samp(j, c2_):
            l = labseg_v[base + j, pl.ds(0, _L)][0]
            do_update(l, base + j)
            return c2_

        lax.fori_loop(0, m, samp, 0)
        return carry

    lax.fori_loop(0, nch, chunk_step, 0)

    drow = pl.multiple_of(o + nch * _L, _L)

    @pl.when((nch & 1) == 0)
    def _drain_even():
        pltpu.make_async_copy(feat_hbm.at[pl.ds(drow, _L)],
                              fbuf_v.at[pl.ds(0, _L)], semf0).wait()
        pltpu.make_async_copy(labbc_hbm.at[pl.ds(drow, _L)],
                              labseg_v.at[pl.ds(0, _L)], semf0).wait()

    @pl.when((nch & 1) == 1)
    def _drain_odd():
        pltpu.make_async_copy(feat_hbm.at[pl.ds(drow, _L)],
                              fbuf_v.at[pl.ds(_L, _L)], semf1).wait()
        pltpu.make_async_copy(labbc_hbm.at[pl.ds(drow, _L)],
                              labseg_v.at[pl.ds(_L, _L)], semf1).wait()

    @pl.when(wid < _NW - 1)
    def _full_out():
        pltpu.sync_copy(blk_v, out_hbm.at[pl.ds(lo, _RPW)])
        pltpu.sync_copy(scl_v, scl_hbm.at[pl.ds(lo, _RPW)])

    @pl.when(wid == _NW - 1)
    def _tail_out():
        pltpu.sync_copy(blk_v.at[pl.ds(0, _TAIL)],
                        out_hbm.at[pl.ds(lo, _TAIL)])
        pltpu.sync_copy(scl_v.at[pl.ds(0, _TAIL)],
                        scl_hbm.at[pl.ds(lo, _TAIL)])


def _sc_update(labbc, ocnt, fperm, prototypes):
    mesh = plsc.VectorSubcoreMesh(core_axis_name="c", subcore_axis_name="s")
    return pl.kernel(
        _sc_body,
        out_type=[
            jax.ShapeDtypeStruct((N_CLS, FEAT_DIM), jnp.float32),
            jax.ShapeDtypeStruct((N_CLS, _L), jnp.float32),
        ],
        mesh=mesh,
        scratch_types=[
            pltpu.VMEM((_L,), jnp.int32),
            pltpu.VMEM((_RPW, FEAT_DIM), jnp.float32),
            pltpu.VMEM((2 * _L, FEAT_DIM), jnp.float32),
            pltpu.VMEM((2 * _L, _L), jnp.int32),
            pltpu.VMEM((_RPW, _L), jnp.float32),
            pltpu.SemaphoreType.DMA,
            pltpu.SemaphoreType.DMA,
            pltpu.SemaphoreType.DMA,
        ],
    )(labbc, ocnt.reshape(-1), fperm, prototypes)


def _loss_body(proto_ref, scl_ref, out_ref):
    g = scl_ref[...][:, 0:1]
    protos = proto_ref[...] * g
    logits = lax.dot_general(
        protos, protos, (((1,), (1,)), ((), ())),
        preferred_element_type=jnp.float32) * (1.0 / TEMPERATURE)
    e = jnp.exp(logits)
    ri = lax.broadcasted_iota(jnp.int32, (N_CLS, N_CLS), 0)
    ci = lax.broadcasted_iota(jnp.int32, (N_CLS, N_CLS), 1)
    e = jnp.where(ri == ci, 0.0, e)
    s = jnp.sum(e, axis=1)
    mpn = jnp.log(s * (1.0 / (N_CLS - 1)))
    nan = jnp.isnan(mpn)
    total = jnp.sum(jnp.where(nan, 0.0, mpn))
    cnt = jnp.sum(jnp.where(nan, 0.0, 1.0))
    out_ref[0, 0] = (TEMPERATURE / BASE_TEMPERATURE) * total / cnt


def kernel(features, labels, prototypes):
    labels = labels.astype(jnp.int32)
    fperm, labbc, ocnt = _prep(labels, features)
    upd, scl = _sc_update(labbc, ocnt, fperm, prototypes)
    out = pl.pallas_call(
        _loss_body,
        out_shape=jax.ShapeDtypeStruct((1, 1), jnp.float32),
        in_specs=[pl.BlockSpec(memory_space=pltpu.VMEM),
                  pl.BlockSpec(memory_space=pltpu.VMEM)],
        out_specs=pl.BlockSpec(memory_space=pltpu.SMEM),
    )(upd, scl)
    return out[0, 0]
